# baseline (speedup 1.0000x reference)
import functools

import jax
import jax.numpy as jnp
from jax import lax
from jax.experimental import pallas as pl
from jax.experimental.pallas import tpu as pltpu
from jax.experimental.pallas import tpu_sc as plsc

CHUNK = 128
NBUF = 8
GAHEAD = 6


@functools.partial(jax.jit, static_argnames=("n_chunks", "n_workers", "emb"))
def _sc_gather(idx3, table, *, n_chunks, n_workers, emb):
    n_rows = n_workers * n_chunks * CHUNK
    mesh = plsc.VectorSubcoreMesh(core_axis_name="c", subcore_axis_name="s")
    nc = mesh.num_cores
    per_w = n_chunks * CHUNK

    def body(table_hbm, idx_hbm, out_hbm, idx_v, rows_v, gsem, wsem):
        wid = lax.axis_index("s") * nc + lax.axis_index("c")
        base = wid * per_w
        pltpu.sync_copy(idx_hbm.at[wid], idx_v)

        @pl.loop(0, n_chunks + GAHEAD)
        def _(j):
            @pl.when(j < n_chunks)
            def _():
                b = lax.rem(j, NBUF)

                @pl.when(j >= NBUF)
                def _():
                    pltpu.make_async_copy(
                        rows_v.at[b],
                        out_hbm.at[pl.ds(base + (j - NBUF) * CHUNK, CHUNK)],
                        wsem,
                    ).wait()

                pltpu.async_copy(table_hbm.at[idx_v.at[j]], rows_v.at[b], gsem)

            @pl.when(j >= GAHEAD)
            def _():
                jj = j - GAHEAD
                bb = lax.rem(jj, NBUF)
                pltpu.make_async_copy(
                    table_hbm.at[idx_v.at[jj]], rows_v.at[bb], gsem
                ).wait()
                pltpu.async_copy(
                    rows_v.at[bb],
                    out_hbm.at[pl.ds(base + jj * CHUNK, CHUNK)],
                    wsem,
                )

        @pl.loop(0, NBUF)
        def _(t):
            jj = n_chunks - NBUF + t
            bb = lax.rem(jj, NBUF)
            pltpu.make_async_copy(
                rows_v.at[bb],
                out_hbm.at[pl.ds(base + jj * CHUNK, CHUNK)],
                wsem,
            ).wait()

    run = pl.kernel(
        body,
        out_type=jax.ShapeDtypeStruct((n_rows, emb), jnp.float32),
        mesh=mesh,
        compiler_params=pltpu.CompilerParams(use_tc_tiling_on_sc=False),
        scratch_types=[
            pltpu.VMEM((n_chunks, CHUNK), jnp.int32),
            pltpu.VMEM((NBUF, CHUNK, emb), jnp.float32),
            pltpu.SemaphoreType.DMA,
            pltpu.SemaphoreType.DMA,
        ],
    )
    return run(table, idx3)


def kernel(inputs, shared_weights):
    bsz, length = inputs.shape
    vocab, emb = shared_weights.shape
    n = bsz * length
    info = plsc.get_sparse_core_info()
    n_workers = info.num_cores * info.num_subcores
    n_chunks = n // (n_workers * CHUNK)
    assert n_chunks * n_workers * CHUNK == n
    idx3 = inputs.reshape(n_workers, n_chunks, CHUNK).astype(jnp.int32)
    out = _sc_gather(
        idx3, shared_weights, n_chunks=n_chunks, n_workers=n_workers, emb=emb
    )
    return out.reshape(bsz, length, emb)


# speedup vs baseline: 1.0004x; 1.0004x over previous
import functools

import jax
import jax.numpy as jnp
from jax import lax
from jax.experimental import pallas as pl
from jax.experimental.pallas import tpu as pltpu
from jax.experimental.pallas import tpu_sc as plsc

NBUF = 6
GAHEAD = 4


@functools.partial(jax.jit, static_argnames=("rows_per_w", "length", "emb"))
def _sc_gather(idx, table, *, rows_per_w, length, emb):
    bsz = idx.shape[0]
    mesh = plsc.VectorSubcoreMesh(core_axis_name="c", subcore_axis_name="s")
    nc = mesh.num_cores
    c0 = 128 if length > 128 else length
    c1 = length - c0

    def body(table_hbm, idx_hbm, out_hbm, idx_v, rows_v, gsem, wsem):
        wid = lax.axis_index("s") * nc + lax.axis_index("c")
        base = wid * rows_per_w
        pltpu.sync_copy(idx_hbm.at[pl.ds(base, rows_per_w)], idx_v)

        def fire_gather(row, buf):
            pltpu.async_copy(
                table_hbm.at[idx_v.at[row, pl.ds(0, c0)]],
                rows_v.at[buf, pl.ds(0, c0)],
                gsem,
            )
            if c1:
                pltpu.async_copy(
                    table_hbm.at[idx_v.at[row, pl.ds(c0, c1)]],
                    rows_v.at[buf, pl.ds(c0, c1)],
                    gsem,
                )

        def wait_gather(row, buf):
            pltpu.make_async_copy(
                table_hbm.at[idx_v.at[row, pl.ds(0, c0)]],
                rows_v.at[buf, pl.ds(0, c0)],
                gsem,
            ).wait()
            if c1:
                pltpu.make_async_copy(
                    table_hbm.at[idx_v.at[row, pl.ds(c0, c1)]],
                    rows_v.at[buf, pl.ds(c0, c1)],
                    gsem,
                ).wait()

        def write_desc(row, buf):
            return pltpu.make_async_copy(
                rows_v.at[buf], out_hbm.at[base + row], wsem
            )

        @pl.loop(0, rows_per_w + GAHEAD)
        def _(j):
            @pl.when(j < rows_per_w)
            def _():
                b = lax.rem(j, NBUF)

                @pl.when(j >= NBUF)
                def _():
                    write_desc(j - NBUF, b).wait()

                fire_gather(j, b)

            @pl.when(j >= GAHEAD)
            def _():
                jj = j - GAHEAD
                bb = lax.rem(jj, NBUF)
                wait_gather(jj, bb)
                write_desc(jj, bb).start()

        @pl.loop(0, NBUF)
        def _(t):
            jj = rows_per_w - NBUF + t
            write_desc(jj, lax.rem(jj, NBUF)).wait()

    run = pl.kernel(
        body,
        out_type=jax.ShapeDtypeStruct((bsz, length, emb), jnp.float32),
        mesh=mesh,
        compiler_params=pltpu.CompilerParams(use_tc_tiling_on_sc=False),
        scratch_types=[
            pltpu.VMEM((rows_per_w, length), jnp.int32),
            pltpu.VMEM((NBUF, length, emb), jnp.float32),
            pltpu.SemaphoreType.DMA,
            pltpu.SemaphoreType.DMA,
        ],
    )
    return run(table, idx)


def kernel(inputs, shared_weights):
    bsz, length = inputs.shape
    vocab, emb = shared_weights.shape
    info = plsc.get_sparse_core_info()
    n_workers = info.num_cores * info.num_subcores
    rows_per_w = bsz // n_workers
    assert rows_per_w * n_workers == bsz
    idx = inputs if inputs.dtype == jnp.int32 else inputs.astype(jnp.int32)
    return _sc_gather(
        idx, shared_weights, rows_per_w=rows_per_w, length=length, emb=emb
    )
